# Initial kernel scaffold; baseline (speedup 1.0000x reference)
import jax
import jax.numpy as jnp
from jax.experimental import pallas as pl


def kernel(x, W_enc, b_enc, emb, W_dec, b_dec):
    raise NotImplementedError("write your pallas kernel here")


# speedup vs baseline: 3.7029x; 3.7029x over previous
import jax
import jax.numpy as jnp
from jax.experimental import pallas as pl
from jax.experimental.pallas import tpu as pltpu

IN_CH = 64
LATENT = 32
K = 8192
T = 1024
TH = T // 2
KB = 2048
NK = K // KB
_PREC = jax.lax.Precision.HIGHEST
_INT_MAX = 2**31 - 1


def _enc_argmin_body(x4_ref, wc_ref, be_ref, emb_ref, ze_ref, idx_ref, z_s, bv_s, bi_s):
    j = pl.program_id(0)

    @pl.when(j == 0)
    def _():
        z = jax.lax.dot_general(wc_ref[...], x4_ref[...], (((1,), (0,)), ((), ())),
                                precision=jax.lax.Precision.DEFAULT,
                                preferred_element_type=jnp.float32)
        z = z + be_ref[...]
        z_s[...] = z
        ze_ref[...] = z

    e = emb_ref[...]
    g = jax.lax.dot_general(e, z_s[...], (((1,), (0,)), ((), ())),
                            precision=_PREC, preferred_element_type=jnp.float32)
    en = jnp.sum(e * e, axis=1, keepdims=True)
    scores = en - 2.0 * g
    lmin = jnp.min(scores, axis=0, keepdims=True)
    rowi = jax.lax.broadcasted_iota(jnp.int32, scores.shape, 0) + j * KB
    lidx = jnp.min(jnp.where(scores == lmin, rowi, _INT_MAX), axis=0, keepdims=True)

    @pl.when(j == 0)
    def _():
        bv_s[...] = lmin
        bi_s[...] = lidx

    @pl.when(j > 0)
    def _():
        better = lmin < bv_s[...]
        bv_s[...] = jnp.where(better, lmin, bv_s[...])
        bi_s[...] = jnp.where(better, lidx, bi_s[...])

    @pl.when(j == NK - 1)
    def _():
        idx_ref[...] = bi_s[...]


def _gather_decode_body(emb_ref, idx_ref, wd_ref, bd_ref, zq_ref, ev_ref, od_ref, acc_s):
    j = pl.program_id(0)
    rowi = jax.lax.broadcasted_iota(jnp.int32, (KB, TH), 0) + j * KB
    oh = jnp.where(rowi == idx_ref[...], 1.0, 0.0)
    part = jax.lax.dot_general(emb_ref[...], oh, (((0,), (0,)), ((), ())),
                               precision=_PREC, preferred_element_type=jnp.float32)

    @pl.when(j == 0)
    def _():
        acc_s[...] = part

    @pl.when(j > 0)
    def _():
        acc_s[...] = acc_s[...] + part

    @pl.when(j == NK - 1)
    def _():
        zq = acc_s[...]
        zq_ref[...] = zq
        r = jax.lax.dot_general(wd_ref[...], zq, (((1,), (0,)), ((), ())),
                                precision=_PREC, preferred_element_type=jnp.float32)
        r0 = r[0:64, :]
        r1 = r[64:128, :]
        r2 = r[128:192, :]
        r3 = r[192:256, :]
        zpad = jnp.zeros((64, 1), jnp.float32)
        shr = jnp.concatenate([zpad, r0[:, :-1]], axis=1)
        shl = jnp.concatenate([r3[:, 1:], zpad], axis=1)
        ev_ref[...] = shr + r2 + bd_ref[...]
        od_ref[...] = r1 + shl + bd_ref[...]


def kernel(x, W_enc, b_enc, emb, W_dec, b_dec):
    E = x[:, 0::2]
    O = x[:, 1::2]
    zc = jnp.zeros((IN_CH, 1), jnp.float32)
    Osh = jnp.concatenate([zc, O[:, :-1]], axis=1)
    Esh = jnp.concatenate([E[:, 1:], zc], axis=1)
    X4 = jnp.stack([Osh, E, O, Esh], axis=1).reshape(4 * IN_CH, TH)
    Wcat = W_enc.reshape(LATENT, 4 * IN_CH)
    Wd2 = jnp.concatenate([W_dec[:, :, 0], W_dec[:, :, 1],
                           W_dec[:, :, 2], W_dec[:, :, 3]], axis=0)

    z_e, idx2 = pl.pallas_call(
        _enc_argmin_body,
        grid=(NK,),
        in_specs=[
            pl.BlockSpec((4 * IN_CH, TH), lambda j: (0, 0)),
            pl.BlockSpec((LATENT, 4 * IN_CH), lambda j: (0, 0)),
            pl.BlockSpec((LATENT, 1), lambda j: (0, 0)),
            pl.BlockSpec((KB, LATENT), lambda j: (j, 0)),
        ],
        out_specs=[
            pl.BlockSpec((LATENT, TH), lambda j: (0, 0)),
            pl.BlockSpec((1, TH), lambda j: (0, 0)),
        ],
        out_shape=[
            jax.ShapeDtypeStruct((LATENT, TH), jnp.float32),
            jax.ShapeDtypeStruct((1, TH), jnp.int32),
        ],
        scratch_shapes=[
            pltpu.VMEM((LATENT, TH), jnp.float32),
            pltpu.VMEM((1, TH), jnp.float32),
            pltpu.VMEM((1, TH), jnp.int32),
        ],
    )(X4, Wcat, b_enc, emb)

    z_q, ev, od = pl.pallas_call(
        _gather_decode_body,
        grid=(NK,),
        in_specs=[
            pl.BlockSpec((KB, LATENT), lambda j: (j, 0)),
            pl.BlockSpec((1, TH), lambda j: (0, 0)),
            pl.BlockSpec((4 * IN_CH, LATENT), lambda j: (0, 0)),
            pl.BlockSpec((IN_CH, 1), lambda j: (0, 0)),
        ],
        out_specs=[
            pl.BlockSpec((LATENT, TH), lambda j: (0, 0)),
            pl.BlockSpec((IN_CH, TH), lambda j: (0, 0)),
            pl.BlockSpec((IN_CH, TH), lambda j: (0, 0)),
        ],
        out_shape=[
            jax.ShapeDtypeStruct((LATENT, TH), jnp.float32),
            jax.ShapeDtypeStruct((IN_CH, TH), jnp.float32),
            jax.ShapeDtypeStruct((IN_CH, TH), jnp.float32),
        ],
        scratch_shapes=[
            pltpu.VMEM((LATENT, TH), jnp.float32),
        ],
    )(emb, idx2, Wd2, b_dec)

    x_recon = jnp.stack([ev, od], axis=-1).reshape(IN_CH, T)
    indices = idx2.reshape(TH)
    return (x_recon, z_e, z_q, indices)
